# Initial kernel scaffold; baseline (speedup 1.0000x reference)
import jax
import jax.numpy as jnp
from jax.experimental import pallas as pl


def kernel(x, char_table, syllable_table, converter):
    raise NotImplementedError("write your pallas kernel here")


# speedup vs baseline: 18.8987x; 18.8987x over previous
import functools

import jax
import jax.numpy as jnp
from jax import lax
from jax.experimental import pallas as pl
from jax.experimental.pallas import tpu as pltpu
from jax.experimental.pallas import tpu_sc as plsc

NC, NS, L = 2, 16, 16
NW = NC * NS
CHUNK = 128
NSLOT = 4
LOOKAHEAD = 2

_CP = pltpu.CompilerParams(use_tc_tiling_on_sc=False)

_mesh = plsc.VectorSubcoreMesh(core_axis_name="c", subcore_axis_name="s",
                               num_cores=NC, num_subcores=NS)


def _worker_id():
    return lax.axis_index("s") * NC + lax.axis_index("c")


def _build_combined(char_table, syllable_table, converter):
    n_rows, d = char_table.shape
    n_full = n_rows // CHUNK
    rem = n_rows - n_full * CHUNK
    per_small = n_full // NW
    n_big = n_full % NW

    @functools.partial(
        pl.kernel,
        out_type=jax.ShapeDtypeStruct((n_rows, d), jnp.float32),
        mesh=_mesh,
        scratch_types=[
            pltpu.VMEM(((per_small + 1) * CHUNK,), jnp.int32),
            pltpu.VMEM((CHUNK, d), jnp.float32),
            pltpu.VMEM((CHUNK, d), jnp.float32),
            pltpu.SemaphoreType.DMA,
        ],
        compiler_params=_CP,
    )
    def build(char_hbm, syll_hbm, conv_hbm, comb_hbm, conv_v, sv, acc, sem):
        wid = _worker_id()
        start = wid * per_small + jnp.minimum(wid, n_big)
        extra = wid < n_big

        pltpu.sync_copy(conv_hbm.at[pl.ds(start * CHUNK, per_small * CHUNK)],
                        conv_v.at[pl.ds(0, per_small * CHUNK)])

        @pl.when(extra)
        def _():
            pltpu.sync_copy(
                conv_hbm.at[pl.ds((start + per_small) * CHUNK, CHUNK)],
                conv_v.at[pl.ds(per_small * CHUNK, CHUNK)])

        def do_chunk(base, conv_off, size):
            cp = pltpu.async_copy(
                syll_hbm.at[conv_v.at[pl.ds(conv_off, size)]],
                sv.at[pl.ds(0, size)], sem)
            pltpu.sync_copy(char_hbm.at[pl.ds(base, size)],
                            acc.at[pl.ds(0, size)])
            cp.wait()

            def row_body(i, _):
                for j in range(d // L):
                    sl = pl.ds(j * L, L)
                    acc[i, sl] = acc[i, sl] + sv[i, sl]
                return 0
            lax.fori_loop(0, size, row_body, 0, unroll=4)

            pltpu.sync_copy(acc.at[pl.ds(0, size)],
                            comb_hbm.at[pl.ds(base, size)])

        def loop_body(t, _):
            do_chunk((start + t) * CHUNK, t * CHUNK, CHUNK)
            return 0

        lax.fori_loop(0, per_small, loop_body, 0)

        @pl.when(extra)
        def _():
            do_chunk((start + per_small) * CHUNK, per_small * CHUNK, CHUNK)

        if rem:
            @pl.when(wid == NW - 1)
            def _():
                pltpu.sync_copy(conv_hbm.at[pl.ds(n_full * CHUNK, rem)],
                                conv_v.at[pl.ds(0, rem)])
                do_chunk(n_full * CHUNK, 0, rem)

    return build(char_table, syllable_table, converter)


def _lookup(combined, x2d):
    n_rows, d = combined.shape
    n_chunks = x2d.shape[0]
    per_w = n_chunks // NW

    @functools.partial(
        pl.kernel,
        out_type=jax.ShapeDtypeStruct((n_chunks * CHUNK, d), jnp.float32),
        mesh=_mesh,
        scratch_types=[
            pltpu.VMEM((per_w, CHUNK), jnp.int32),
            pltpu.VMEM((NSLOT, CHUNK, d), jnp.float32),
            pltpu.SemaphoreType.DMA((NSLOT,)),
            pltpu.SemaphoreType.DMA((NSLOT,)),
        ],
        compiler_params=_CP,
    )
    def lookup(comb_hbm, idx_hbm, out_hbm, idx_v, rows, gsem, osem):
        wid = _worker_id()
        c0 = wid * per_w

        pltpu.sync_copy(idx_hbm.at[pl.ds(c0, per_w)], idx_v)

        def gather(t, slot):
            return pltpu.async_copy(comb_hbm.at[idx_v.at[t]], rows.at[slot],
                                    gsem.at[slot])

        def writeback(t, slot):
            return pltpu.async_copy(rows.at[slot],
                                    out_hbm.at[pl.ds((c0 + t) * CHUNK, CHUNK)],
                                    osem.at[slot])

        def wait_gather(slot):
            pltpu.make_async_copy(comb_hbm.at[pl.ds(0, CHUNK)], rows.at[slot],
                                  gsem.at[slot]).wait()

        def wait_writeback(slot):
            pltpu.make_async_copy(rows.at[slot],
                                  out_hbm.at[pl.ds(0, CHUNK)],
                                  osem.at[slot]).wait()

        for b in range(LOOKAHEAD):
            gather(b, b)

        def group(g, _):
            for b in range(NSLOT):
                t = g * NSLOT + b
                wait_gather(b)
                writeback(t, b)
                u = t + LOOKAHEAD
                s = (b + LOOKAHEAD) % NSLOT

                @pl.when(u < per_w)
                def _():
                    @pl.when(u >= NSLOT)
                    def _():
                        wait_writeback(s)
                    gather(u, s)
            return 0

        lax.fori_loop(0, per_w // NSLOT, group, 0)

        for b in range(NSLOT):
            wait_writeback(b)

    return lookup(combined, x2d)


def kernel(x, char_table, syllable_table, converter):
    xshape = x.shape
    d = char_table.shape[1]
    x2d = x.reshape(-1, CHUNK).astype(jnp.int32)
    combined = _build_combined(char_table, syllable_table,
                               converter.astype(jnp.int32))
    out = _lookup(combined, x2d)
    return out.reshape(*xshape, d)


# speedup vs baseline: 23.5567x; 1.2465x over previous
import functools

import jax
import jax.numpy as jnp
from jax import lax
from jax.experimental import pallas as pl
from jax.experimental.pallas import tpu as pltpu
from jax.experimental.pallas import tpu_sc as plsc

NC, NS, L = 2, 16, 16
NW = NC * NS
CHUNK = 128
NSLOT = 4
LOOKAHEAD = 2

_CP = pltpu.CompilerParams(use_tc_tiling_on_sc=True)

_mesh = plsc.VectorSubcoreMesh(core_axis_name="c", subcore_axis_name="s",
                               num_cores=NC, num_subcores=NS)


def _worker_id():
    return lax.axis_index("s") * NC + lax.axis_index("c")


def _build_combined(char_table, syll_pad, conv_pad, n_rows):
    d = char_table.shape[1]
    dp = syll_pad.shape[1]
    n_full = n_rows // CHUNK
    rem = n_rows - n_full * CHUNK
    per_small = n_full // NW
    n_big = n_full % NW

    @functools.partial(
        pl.kernel,
        out_type=jax.ShapeDtypeStruct((n_rows, dp), jnp.float32),
        mesh=_mesh,
        scratch_types=[
            pltpu.VMEM(((per_small + 2) * CHUNK,), jnp.int32),
            pltpu.VMEM((CHUNK, dp), jnp.float32),
            pltpu.VMEM((CHUNK, d), jnp.float32),
            pltpu.SemaphoreType.DMA,
        ],
        compiler_params=_CP,
    )
    def build(char_hbm, syll_hbm, conv_hbm, comb_hbm, conv_v, sv, acc, sem):
        wid = _worker_id()
        start = wid * per_small + jnp.minimum(wid, n_big)
        extra = wid < n_big

        pltpu.sync_copy(conv_hbm.at[pl.ds(start * CHUNK, per_small * CHUNK)],
                        conv_v.at[pl.ds(0, per_small * CHUNK)])

        @pl.when(extra)
        def _():
            pltpu.sync_copy(
                conv_hbm.at[pl.ds((start + per_small) * CHUNK, CHUNK)],
                conv_v.at[pl.ds(per_small * CHUNK, CHUNK)])

        def do_chunk(base, conv_off, size):
            cp = pltpu.async_copy(
                syll_hbm.at[conv_v.at[pl.ds(conv_off, CHUNK)]],
                sv, sem)
            pltpu.sync_copy(char_hbm.at[pl.ds(base, size)],
                            acc.at[pl.ds(0, size)])
            cp.wait()

            def row_body(i, _):
                for j in range(d // L):
                    sl = pl.ds(j * L, L)
                    sv[i, sl] = acc[i, sl] + sv[i, sl]
                return 0
            lax.fori_loop(0, size, row_body, 0, unroll=4)

            pltpu.sync_copy(sv.at[pl.ds(0, size)],
                            comb_hbm.at[pl.ds(base, size)])

        def loop_body(t, _):
            do_chunk((start + t) * CHUNK, t * CHUNK, CHUNK)
            return 0

        lax.fori_loop(0, per_small, loop_body, 0)

        @pl.when(extra)
        def _():
            do_chunk((start + per_small) * CHUNK, per_small * CHUNK, CHUNK)

        if rem:
            @pl.when(wid == NW - 1)
            def _():
                pltpu.sync_copy(
                    conv_hbm.at[pl.ds(n_full * CHUNK, CHUNK)],
                    conv_v.at[pl.ds((per_small + 1) * CHUNK, CHUNK)])
                do_chunk(n_full * CHUNK, (per_small + 1) * CHUNK, rem)

    return build(char_table, syll_pad, conv_pad)


def _lookup(combined, x2d, d):
    n_rows, dp = combined.shape
    n_chunks = x2d.shape[0]
    per_w = n_chunks // NW

    @functools.partial(
        pl.kernel,
        out_type=jax.ShapeDtypeStruct((n_chunks * CHUNK, d), jnp.float32),
        mesh=_mesh,
        scratch_types=[
            pltpu.VMEM((per_w, CHUNK), jnp.int32),
            pltpu.VMEM((NSLOT, CHUNK, dp), jnp.float32),
            pltpu.VMEM((2, CHUNK, d), jnp.float32),
            pltpu.SemaphoreType.DMA((NSLOT,)),
            pltpu.SemaphoreType.DMA((2,)),
        ],
        compiler_params=_CP,
    )
    def lookup(comb_hbm, idx_hbm, out_hbm, idx_v, rows, slim, gsem, osem):
        wid = _worker_id()
        c0 = wid * per_w

        pltpu.sync_copy(idx_hbm.at[pl.ds(c0, per_w)], idx_v)

        def gather(t, slot):
            return pltpu.async_copy(comb_hbm.at[idx_v.at[t]], rows.at[slot],
                                    gsem.at[slot])

        def compact(slot, sslot):
            def row_body(i, _):
                for j in range(d // L):
                    sl = pl.ds(j * L, L)
                    slim[sslot, i, sl] = rows[slot, i, sl]
                return 0
            lax.fori_loop(0, CHUNK, row_body, 0, unroll=4)

        def writeback(t, sslot):
            return pltpu.async_copy(slim.at[sslot],
                                    out_hbm.at[pl.ds((c0 + t) * CHUNK, CHUNK)],
                                    osem.at[sslot])

        def wait_gather(slot):
            pltpu.make_async_copy(comb_hbm.at[pl.ds(0, CHUNK)], rows.at[slot],
                                  gsem.at[slot]).wait()

        def wait_writeback(sslot):
            pltpu.make_async_copy(slim.at[sslot],
                                  out_hbm.at[pl.ds(0, CHUNK)],
                                  osem.at[sslot]).wait()

        for b in range(LOOKAHEAD):
            gather(b, b)

        def group(g, _):
            for b in range(NSLOT):
                t = g * NSLOT + b
                sb = b % 2
                wait_gather(b)
                u = t + LOOKAHEAD
                s = (b + LOOKAHEAD) % NSLOT

                @pl.when(u < per_w)
                def _():
                    gather(u, s)

                if b >= 2:
                    wait_writeback(sb)
                else:
                    @pl.when(g > 0)
                    def _():
                        wait_writeback(sb)
                compact(b, sb)
                writeback(t, sb)
            return 0

        lax.fori_loop(0, per_w // NSLOT, group, 0)

        for sb in range(2):
            wait_writeback(sb)

    return lookup(combined, x2d)


def kernel(x, char_table, syllable_table, converter):
    xshape = x.shape
    n_rows, d = char_table.shape
    dp = 2 * d
    x2d = x.reshape(-1, CHUNK).astype(jnp.int32)
    syll_pad = jnp.pad(syllable_table, ((0, 0), (0, dp - d)))
    pad_n = (-n_rows) % CHUNK
    conv_pad = jnp.pad(converter.astype(jnp.int32), (0, pad_n))
    combined = _build_combined(char_table, syll_pad, conv_pad, n_rows)
    out = _lookup(combined, x2d, d)
    return out.reshape(*xshape, d)


# speedup vs baseline: 23.5699x; 1.0006x over previous
import functools

import jax
import jax.numpy as jnp
from jax import lax
from jax.experimental import pallas as pl
from jax.experimental.pallas import tpu as pltpu
from jax.experimental.pallas import tpu_sc as plsc

NC, NS, L = 2, 16, 16
NW = NC * NS
CHUNK = 128
NSLOT = 4
LOOKAHEAD = 2

_CP = pltpu.CompilerParams(use_tc_tiling_on_sc=True)

_mesh = plsc.VectorSubcoreMesh(core_axis_name="c", subcore_axis_name="s",
                               num_cores=NC, num_subcores=NS)


def _worker_id():
    return lax.axis_index("s") * NC + lax.axis_index("c")


def _build_combined(char_table, syll_pad, conv_pad, n_rows):
    d = char_table.shape[1]
    dp = syll_pad.shape[1]
    n_full = n_rows // CHUNK
    rem = n_rows - n_full * CHUNK
    per_small = n_full // NW
    n_big = n_full % NW

    @functools.partial(
        pl.kernel,
        out_type=jax.ShapeDtypeStruct((n_rows, dp), jnp.float32),
        mesh=_mesh,
        scratch_types=[
            pltpu.VMEM(((per_small + 2) * CHUNK,), jnp.int32),
            pltpu.VMEM((CHUNK, dp), jnp.float32),
            pltpu.VMEM((CHUNK, d), jnp.float32),
            pltpu.SemaphoreType.DMA,
        ],
        compiler_params=_CP,
    )
    def build(char_hbm, syll_hbm, conv_hbm, comb_hbm, conv_v, sv, acc, sem):
        wid = _worker_id()
        start = wid * per_small + jnp.minimum(wid, n_big)
        extra = wid < n_big

        pltpu.sync_copy(conv_hbm.at[pl.ds(start * CHUNK, per_small * CHUNK)],
                        conv_v.at[pl.ds(0, per_small * CHUNK)])

        @pl.when(extra)
        def _():
            pltpu.sync_copy(
                conv_hbm.at[pl.ds((start + per_small) * CHUNK, CHUNK)],
                conv_v.at[pl.ds(per_small * CHUNK, CHUNK)])

        def do_chunk(base, conv_off, size):
            cp = pltpu.async_copy(
                syll_hbm.at[conv_v.at[pl.ds(conv_off, CHUNK)]],
                sv, sem)
            pltpu.sync_copy(char_hbm.at[pl.ds(base, size)],
                            acc.at[pl.ds(0, size)])
            cp.wait()

            def row_body(i, _):
                for j in range(d // L):
                    sl = pl.ds(j * L, L)
                    sv[i, sl] = acc[i, sl] + sv[i, sl]
                return 0
            lax.fori_loop(0, size, row_body, 0, unroll=4)

            pltpu.sync_copy(sv.at[pl.ds(0, size)],
                            comb_hbm.at[pl.ds(base, size)])

        def loop_body(t, _):
            do_chunk((start + t) * CHUNK, t * CHUNK, CHUNK)
            return 0

        lax.fori_loop(0, per_small, loop_body, 0)

        @pl.when(extra)
        def _():
            do_chunk((start + per_small) * CHUNK, per_small * CHUNK, CHUNK)

        if rem:
            @pl.when(wid == NW - 1)
            def _():
                pltpu.sync_copy(
                    conv_hbm.at[pl.ds(n_full * CHUNK, CHUNK)],
                    conv_v.at[pl.ds((per_small + 1) * CHUNK, CHUNK)])
                do_chunk(n_full * CHUNK, (per_small + 1) * CHUNK, rem)

    return build(char_table, syll_pad, conv_pad)


def _lookup(combined, x2d, d):
    n_rows, dp = combined.shape
    n_chunks = x2d.shape[0]
    per_w = n_chunks // NW

    @functools.partial(
        pl.kernel,
        out_type=jax.ShapeDtypeStruct((n_chunks, CHUNK, d), jnp.float32),
        mesh=_mesh,
        scratch_types=[
            pltpu.VMEM((per_w, CHUNK), jnp.int32),
            pltpu.VMEM((NSLOT, CHUNK, dp), jnp.float32),
            pltpu.VMEM((2, CHUNK, d), jnp.float32),
            pltpu.SemaphoreType.DMA((NSLOT,)),
            pltpu.SemaphoreType.DMA((2,)),
        ],
        compiler_params=_CP,
    )
    def lookup(comb_hbm, idx_hbm, out_hbm, idx_v, rows, slim, gsem, osem):
        wid = _worker_id()
        c0 = wid * per_w

        pltpu.sync_copy(idx_hbm.at[pl.ds(c0, per_w)], idx_v)

        def gather(t, slot):
            return pltpu.async_copy(comb_hbm.at[idx_v.at[t]], rows.at[slot],
                                    gsem.at[slot])

        def compact(slot, sslot):
            def row_body(i, _):
                for j in range(d // L):
                    sl = pl.ds(j * L, L)
                    slim[sslot, i, sl] = rows[slot, i, sl]
                return 0
            lax.fori_loop(0, CHUNK, row_body, 0, unroll=4)

        def writeback(t, sslot):
            return pltpu.async_copy(slim.at[sslot], out_hbm.at[c0 + t],
                                    osem.at[sslot])

        def wait_gather(slot):
            pltpu.make_async_copy(comb_hbm.at[pl.ds(0, CHUNK)], rows.at[slot],
                                  gsem.at[slot]).wait()

        def wait_writeback(sslot):
            pltpu.make_async_copy(slim.at[sslot], out_hbm.at[0],
                                  osem.at[sslot]).wait()

        for b in range(LOOKAHEAD):
            gather(b, b)

        def group(g, _):
            for b in range(NSLOT):
                t = g * NSLOT + b
                sb = b % 2
                wait_gather(b)
                u = t + LOOKAHEAD
                s = (b + LOOKAHEAD) % NSLOT

                @pl.when(u < per_w)
                def _():
                    gather(u, s)

                if b >= 2:
                    wait_writeback(sb)
                else:
                    @pl.when(g > 0)
                    def _():
                        wait_writeback(sb)
                compact(b, sb)
                writeback(t, sb)
            return 0

        lax.fori_loop(0, per_w // NSLOT, group, 0)

        for sb in range(2):
            wait_writeback(sb)

    return lookup(combined, x2d)


def kernel(x, char_table, syllable_table, converter):
    xshape = x.shape
    n_rows, d = char_table.shape
    dp = 2 * d
    x2d = x.reshape(-1, CHUNK).astype(jnp.int32)
    syll_pad = jnp.pad(syllable_table, ((0, 0), (0, dp - d)))
    pad_n = (-n_rows) % CHUNK
    conv_pad = jnp.pad(converter.astype(jnp.int32), (0, pad_n))
    combined = _build_combined(char_table, syll_pad, conv_pad, n_rows)
    out = _lookup(combined, x2d, d)
    return out.reshape(*xshape, d)
